# baseline (speedup 1.0000x reference)
import jax
import jax.numpy as jnp
import numpy as np
from jax import lax
from jax.experimental import pallas as pl
from jax.experimental.pallas import tpu as pltpu

C_IN, C_OUT, KH, KW = 3, 6, 5, 5
H, W = 32, 32
HO, WO = H - KH + 1, W - KW + 1
HW = H * W
N_CLS = 3
KWIN = KH * W
NF = 8 * W
FT = HO * NF
ONE_COL = C_OUT * W
NL = 8
TB = 1024

_WIN = np.arange(W)[:, None]
_WO = np.arange(W)[None, :]
_M = np.stack([((_WIN - _WO) == j) & (_WO < WO) for j in range(KW)])
_MASK = _M.reshape(KW, W, W).astype(np.float32)
_BSPREAD = np.zeros((C_OUT, NF), np.float32)
for _co in range(C_OUT):
    _BSPREAD[_co, _co * W:_co * W + WO] = 1.0
_E_ONECOL = np.zeros((NF,), np.float32)
_E_ONECOL[ONE_COL] = 1.0
_E_ONECOL_FT = np.zeros((FT,), np.float32)
_E_ONECOL_FT[ONE_COL] = 1.0


def _fused_body(x_ref, wc_ref, brow_ref, w2_ref, o_ref, xb_ref, feat_ref):
    xb_ref[...] = x_ref[...].astype(jnp.bfloat16)
    brow = brow_ref[0:1, :]
    for h in range(HO):
        acc = brow
        for c in range(C_IN):
            acc = acc + lax.dot_general(
                xb_ref[:, c * HW + h * W:c * HW + h * W + KWIN], wc_ref[c],
                (((1,), (0,)), ((), ())), preferred_element_type=jnp.float32)
        feat_ref[:, h * NF:(h + 1) * NF] = (
            jnp.maximum(acc, 0.0).astype(jnp.bfloat16))

    logits = lax.dot_general(
        feat_ref[...], w2_ref[...],
        (((1,), (1,)), ((), ())), preferred_element_type=jnp.float32)
    lg = logits[:, :N_CLS]
    s = lg - jnp.max(lg, axis=-1, keepdims=True)
    o_ref[...] = s - jnp.log(jnp.sum(jnp.exp(s), axis=-1, keepdims=True))


def _build_conv_weights(w1):
    t = jnp.einsum('ocdj,jpq->cdpoq', w1.astype(jnp.float32), _MASK)
    t = t.reshape(C_IN, KWIN, C_OUT * W)
    return jnp.pad(t, ((0, 0), (0, 0), (0, NF - C_OUT * W))).astype(jnp.bfloat16)


def _build_bias_row(b1):
    brow = b1.astype(jnp.float32) @ _BSPREAD + _E_ONECOL
    return jnp.broadcast_to(brow[None, :], (8, NF))


def _build_linear_weights(w2, b2):
    w2r = w2.astype(jnp.float32).reshape(N_CLS, C_OUT, HO, WO)
    w2t = jnp.transpose(w2r, (0, 2, 1, 3))
    w2t = jnp.pad(w2t, ((0, NL - N_CLS), (0, 0), (0, 2), (0, W - WO)))
    w2f = w2t.reshape(NL, FT)
    b2p = jnp.pad(b2.astype(jnp.float32), (0, NL - N_CLS))
    return (w2f + b2p[:, None] * _E_ONECOL_FT[None, :]).astype(jnp.bfloat16)


@jax.jit
def _forward(x, w1, b1, w2, b2):
    n = x.shape[0]
    tb = min(TB, ((n + 7) // 8) * 8)
    n_pad = (-n) % tb
    n_tiles = (n + n_pad) // tb

    x2 = x.reshape(n, C_IN * HW)
    if n_pad:
        x2 = jnp.pad(x2, ((0, n_pad), (0, 0)))

    wc = _build_conv_weights(w1)
    brow = _build_bias_row(b1)
    w2f = _build_linear_weights(w2, b2)

    out = pl.pallas_call(
        _fused_body,
        out_shape=jax.ShapeDtypeStruct((n + n_pad, N_CLS), jnp.float32),
        grid=(n_tiles,),
        in_specs=[
            pl.BlockSpec((tb, C_IN * HW), lambda b: (b, 0)),
            pl.BlockSpec((C_IN, KWIN, NF), lambda b: (0, 0, 0)),
            pl.BlockSpec((8, NF), lambda b: (0, 0)),
            pl.BlockSpec((NL, FT), lambda b: (0, 0)),
        ],
        out_specs=pl.BlockSpec((tb, N_CLS), lambda b: (b, 0)),
        scratch_shapes=[
            pltpu.VMEM((tb, C_IN * HW), jnp.bfloat16),
            pltpu.VMEM((tb, FT), jnp.bfloat16),
        ],
        compiler_params=pltpu.CompilerParams(
            dimension_semantics=("parallel",)),
    )(x2, wc, brow, w2f)
    return out[:n] if n_pad else out


def kernel(x, w1, b1, w2, b2):
    return _forward(x, w1, b1, w2, b2)


# speedup vs baseline: 1.0082x; 1.0082x over previous
import jax
import jax.numpy as jnp
import numpy as np
from jax import lax
from jax.experimental import pallas as pl
from jax.experimental.pallas import tpu as pltpu

C_IN, C_OUT, KH, KW = 3, 6, 5, 5
H, W = 32, 32
HO, WO = H - KH + 1, W - KW + 1
HW = H * W
N_CLS = 3
KWIN = KH * W
NF = 8 * W
FT = HO * NF
ONE_COL = C_OUT * W
NL = 8
TB = 512

_WIN = np.arange(W)[:, None]
_WO = np.arange(W)[None, :]
_M = np.stack([((_WIN - _WO) == j) & (_WO < WO) for j in range(KW)])
_MASK = _M.reshape(KW, W, W).astype(np.float32)
_BSPREAD = np.zeros((C_OUT, NF), np.float32)
for _co in range(C_OUT):
    _BSPREAD[_co, _co * W:_co * W + WO] = 1.0
_E_ONECOL = np.zeros((NF,), np.float32)
_E_ONECOL[ONE_COL] = 1.0
_E_ONECOL_FT = np.zeros((FT,), np.float32)
_E_ONECOL_FT[ONE_COL] = 1.0


def _fused_body(x_ref, wc_ref, brow_ref, w2_ref, o_ref, xb_ref, feat_ref):
    xb_ref[...] = x_ref[...].astype(jnp.bfloat16)
    brow = brow_ref[0:1, :]
    for h in range(HO):
        acc = brow
        for c in range(C_IN):
            acc = acc + lax.dot_general(
                xb_ref[:, c * HW + h * W:c * HW + h * W + KWIN], wc_ref[c],
                (((1,), (0,)), ((), ())), preferred_element_type=jnp.float32)
        feat_ref[:, h * NF:(h + 1) * NF] = (
            jnp.maximum(acc, 0.0).astype(jnp.bfloat16))

    logits = lax.dot_general(
        feat_ref[...], w2_ref[...],
        (((1,), (1,)), ((), ())), preferred_element_type=jnp.float32)
    lg = logits[:, :N_CLS]
    s = lg - jnp.max(lg, axis=-1, keepdims=True)
    o_ref[...] = s - jnp.log(jnp.sum(jnp.exp(s), axis=-1, keepdims=True))


def _build_conv_weights(w1):
    t = jnp.einsum('ocdj,jpq->cdpoq', w1.astype(jnp.float32), _MASK)
    t = t.reshape(C_IN, KWIN, C_OUT * W)
    return jnp.pad(t, ((0, 0), (0, 0), (0, NF - C_OUT * W))).astype(jnp.bfloat16)


def _build_bias_row(b1):
    brow = b1.astype(jnp.float32) @ _BSPREAD + _E_ONECOL
    return jnp.broadcast_to(brow[None, :], (8, NF))


def _build_linear_weights(w2, b2):
    w2r = w2.astype(jnp.float32).reshape(N_CLS, C_OUT, HO, WO)
    w2t = jnp.transpose(w2r, (0, 2, 1, 3))
    w2t = jnp.pad(w2t, ((0, NL - N_CLS), (0, 0), (0, 2), (0, W - WO)))
    w2f = w2t.reshape(NL, FT)
    b2p = jnp.pad(b2.astype(jnp.float32), (0, NL - N_CLS))
    return (w2f + b2p[:, None] * _E_ONECOL_FT[None, :]).astype(jnp.bfloat16)


@jax.jit
def _forward(x, w1, b1, w2, b2):
    n = x.shape[0]
    tb = min(TB, ((n + 7) // 8) * 8)
    n_pad = (-n) % tb
    n_tiles = (n + n_pad) // tb

    x2 = x.reshape(n, C_IN * HW)
    if n_pad:
        x2 = jnp.pad(x2, ((0, n_pad), (0, 0)))

    wc = _build_conv_weights(w1)
    brow = _build_bias_row(b1)
    w2f = _build_linear_weights(w2, b2)

    out = pl.pallas_call(
        _fused_body,
        out_shape=jax.ShapeDtypeStruct((n + n_pad, N_CLS), jnp.float32),
        grid=(n_tiles,),
        in_specs=[
            pl.BlockSpec((tb, C_IN * HW), lambda b: (b, 0)),
            pl.BlockSpec((C_IN, KWIN, NF), lambda b: (0, 0, 0)),
            pl.BlockSpec((8, NF), lambda b: (0, 0)),
            pl.BlockSpec((NL, FT), lambda b: (0, 0)),
        ],
        out_specs=pl.BlockSpec((tb, N_CLS), lambda b: (b, 0)),
        scratch_shapes=[
            pltpu.VMEM((tb, C_IN * HW), jnp.bfloat16),
            pltpu.VMEM((tb, FT), jnp.bfloat16),
        ],
        compiler_params=pltpu.CompilerParams(
            dimension_semantics=("parallel",)),
    )(x2, wc, brow, w2f)
    return out[:n] if n_pad else out


def kernel(x, w1, b1, w2, b2):
    return _forward(x, w1, b1, w2, b2)


# speedup vs baseline: 1.0481x; 1.0396x over previous
import jax
import jax.numpy as jnp
import numpy as np
from jax import lax
from jax.experimental import pallas as pl
from jax.experimental.pallas import tpu as pltpu

C_IN, C_OUT, KH, KW = 3, 6, 5, 5
H, W = 32, 32
HO, WO = H - KH + 1, W - KW + 1
HW = H * W
N_CLS = 3
KWIN = KH * W
NF = 8 * W
FT = HO * NF
ONE_COL = C_OUT * W
NL = 8
TB = 512

_WIN = np.arange(W)[:, None]
_WO = np.arange(W)[None, :]
_M = np.stack([((_WIN - _WO) == j) & (_WO < WO) for j in range(KW)])
_MASK = _M.reshape(KW, W, W).astype(np.float32)
def _fused_body(x_ref, wc_ref, w2_ref, b1_ref, b2_ref, o_ref, xb_ref, feat_ref):
    xb_ref[...] = x_ref[...].astype(jnp.bfloat16)
    brow = jnp.concatenate(
        [jnp.full((1, W), b1_ref[0, co], jnp.float32) for co in range(C_OUT)]
        + [jnp.zeros((1, NF - C_OUT * W), jnp.float32)], axis=1)
    for h in range(HO):
        acc = brow
        for c in range(C_IN):
            acc = acc + lax.dot_general(
                xb_ref[:, c * HW + h * W:c * HW + h * W + KWIN], wc_ref[c],
                (((1,), (0,)), ((), ())), preferred_element_type=jnp.float32)
        feat_ref[:, h * NF:(h + 1) * NF] = (
            jnp.maximum(acc, 0.0).astype(jnp.bfloat16))

    logits = lax.dot_general(
        feat_ref[...], w2_ref[...],
        (((1,), (1,)), ((), ())), preferred_element_type=jnp.float32)
    b2row = jnp.concatenate(
        [jnp.full((1, 1), b2_ref[0, i], jnp.float32) for i in range(N_CLS)],
        axis=1)
    lg = logits[:, :N_CLS] + b2row
    s = lg - jnp.max(lg, axis=-1, keepdims=True)
    o_ref[...] = s - jnp.log(jnp.sum(jnp.exp(s), axis=-1, keepdims=True))


def _build_conv_weights(w1):
    t = jnp.einsum('ocdj,jpq->cdpoq', w1.astype(jnp.float32), _MASK)
    t = t.reshape(C_IN, KWIN, C_OUT * W)
    return jnp.pad(t, ((0, 0), (0, 0), (0, NF - C_OUT * W))).astype(jnp.bfloat16)


def _build_linear_weights(w2):
    w2r = w2.astype(jnp.float32).reshape(N_CLS, C_OUT, HO, WO)
    w2t = jnp.transpose(w2r, (0, 2, 1, 3))
    w2t = jnp.pad(w2t, ((0, NL - N_CLS), (0, 0), (0, 2), (0, W - WO)))
    return w2t.reshape(NL, FT).astype(jnp.bfloat16)


@jax.jit
def _forward(x, w1, b1, w2, b2):
    n = x.shape[0]
    tb = min(TB, ((n + 7) // 8) * 8)
    n_pad = (-n) % tb
    n_tiles = (n + n_pad) // tb

    x2 = x.reshape(n, C_IN * HW)
    if n_pad:
        x2 = jnp.pad(x2, ((0, n_pad), (0, 0)))

    wc = _build_conv_weights(w1)
    w2f = _build_linear_weights(w2)

    out = pl.pallas_call(
        _fused_body,
        out_shape=jax.ShapeDtypeStruct((n + n_pad, N_CLS), jnp.float32),
        grid=(n_tiles,),
        in_specs=[
            pl.BlockSpec((tb, C_IN * HW), lambda b: (b, 0)),
            pl.BlockSpec((C_IN, KWIN, NF), lambda b: (0, 0, 0)),
            pl.BlockSpec((NL, FT), lambda b: (0, 0)),
            pl.BlockSpec(memory_space=pltpu.MemorySpace.SMEM),
            pl.BlockSpec(memory_space=pltpu.MemorySpace.SMEM),
        ],
        out_specs=pl.BlockSpec((tb, N_CLS), lambda b: (b, 0)),
        scratch_shapes=[
            pltpu.VMEM((tb, C_IN * HW), jnp.bfloat16),
            pltpu.VMEM((tb, FT), jnp.bfloat16),
        ],
        compiler_params=pltpu.CompilerParams(
            dimension_semantics=("parallel",)),
    )(x2, wc, w2f, b1.astype(jnp.float32).reshape(1, C_OUT),
      b2.astype(jnp.float32).reshape(1, N_CLS))
    return out[:n] if n_pad else out


def kernel(x, w1, b1, w2, b2):
    return _forward(x, w1, b1, w2, b2)


# speedup vs baseline: 1.0513x; 1.0030x over previous
import jax
import jax.numpy as jnp
import numpy as np
from jax import lax
from jax.experimental import pallas as pl
from jax.experimental.pallas import tpu as pltpu

C_IN, C_OUT, KH, KW = 3, 6, 5, 5
H, W = 32, 32
HO, WO = H - KH + 1, W - KW + 1
HW = H * W
N_CLS = 3
KWIN = KH * W
NF = 8 * W
FT = HO * NF
ONE_COL = C_OUT * W
NL = 8
TB = 512

_WIN = np.arange(W)[:, None]
_WO = np.arange(W)[None, :]
_M = np.stack([((_WIN - _WO) == j) & (_WO < WO) for j in range(KW)])
_MASK = _M.reshape(KW, W, W).astype(np.float32)
def _fused_body(x_ref, wc_ref, w2_ref, b1_ref, b2_ref, o_ref, xb_ref, feat_ref):
    xb_ref[...] = x_ref[...].astype(jnp.bfloat16)
    brow = jnp.concatenate(
        [jnp.full((1, W), b1_ref[0, co], jnp.float32) for co in range(C_OUT)]
        + [jnp.zeros((1, NF - C_OUT * W), jnp.float32)], axis=1)
    for h in range(HO):
        acc = brow
        for c in range(C_IN):
            acc = acc + lax.dot_general(
                xb_ref[:, c * HW + h * W:c * HW + h * W + KWIN], wc_ref[c],
                (((1,), (0,)), ((), ())), preferred_element_type=jnp.float32)
        feat_ref[:, h * NF:(h + 1) * NF] = (
            jnp.maximum(acc, 0.0).astype(jnp.bfloat16))

    logits = lax.dot_general(
        feat_ref[...], w2_ref[...],
        (((1,), (1,)), ((), ())), preferred_element_type=jnp.float32)
    b2row = jnp.concatenate(
        [jnp.full((1, 1), b2_ref[0, i], jnp.float32) for i in range(N_CLS)],
        axis=1)
    lg = logits[:, :N_CLS] + b2row
    s = lg - jnp.max(lg, axis=-1, keepdims=True)
    o_ref[...] = s - jnp.log(jnp.sum(jnp.exp(s), axis=-1, keepdims=True))


def _build_conv_weights(w1):
    t = jnp.einsum('ocdj,jpq->cdpoq', w1.astype(jnp.float32), _MASK)
    t = t.reshape(C_IN, KWIN, C_OUT * W)
    return jnp.pad(t, ((0, 0), (0, 0), (0, NF - C_OUT * W))).astype(jnp.bfloat16)


def _build_linear_weights(w2):
    w2r = w2.astype(jnp.float32).reshape(N_CLS, C_OUT, HO, WO)
    w2t = jnp.transpose(w2r, (0, 2, 1, 3))
    w2t = jnp.pad(w2t, ((0, NL - N_CLS), (0, 0), (0, 2), (0, W - WO)))
    return w2t.reshape(NL, FT).astype(jnp.bfloat16)


@jax.jit
def _forward(x, w1, b1, w2, b2):
    n = x.shape[0]
    tb = min(TB, ((n + 7) // 8) * 8)
    n_pad = (-n) % tb
    n_tiles = (n + n_pad) // tb

    x2 = x.reshape(n, C_IN * HW)
    if n_pad:
        x2 = jnp.pad(x2, ((0, n_pad), (0, 0)))

    wc = _build_conv_weights(w1)
    w2f = _build_linear_weights(w2)

    out = pl.pallas_call(
        _fused_body,
        out_shape=jax.ShapeDtypeStruct((n + n_pad, N_CLS), jnp.float32),
        grid=(n_tiles,),
        in_specs=[
            pl.BlockSpec((tb, C_IN * HW), lambda b: (b, 0)),
            pl.BlockSpec((C_IN, KWIN, NF), lambda b: (0, 0, 0)),
            pl.BlockSpec((NL, FT), lambda b: (0, 0)),
            pl.BlockSpec(memory_space=pltpu.MemorySpace.SMEM),
            pl.BlockSpec(memory_space=pltpu.MemorySpace.SMEM),
        ],
        out_specs=pl.BlockSpec((tb, N_CLS), lambda b: (b, 0)),
        scratch_shapes=[
            pltpu.VMEM((tb, C_IN * HW), jnp.bfloat16),
            pltpu.VMEM((tb, FT), jnp.bfloat16),
        ],
        compiler_params=pltpu.CompilerParams(
            dimension_semantics=("arbitrary",)),
    )(x2, wc, w2f, b1.astype(jnp.float32).reshape(1, C_OUT),
      b2.astype(jnp.float32).reshape(1, N_CLS))
    return out[:n] if n_pad else out


def kernel(x, w1, b1, w2, b2):
    return _forward(x, w1, b1, w2, b2)
